# baseline (speedup 1.0000x reference)
import functools

import jax
import jax.numpy as jnp
from jax import lax
from jax.experimental import pallas as pl
from jax.experimental.pallas import tpu as pltpu
from jax.experimental.pallas import tpu_sc as plsc

N = 100000
E = 3200000

NC = 2
NS = 16
LANES = 128
MR = 4
NMEG = 8
RPB = NMEG * MR

NPAD = 100352
RPT = NPAD // NS
EPAD = 3276800
ROWS_E = EPAD // LANES
RW1 = ROWS_E // (NC * NS)
RW2 = ROWS_E // NS
NB1 = RW1 // RPB
NB2 = RW2 // RPB
DBR = 40
NDB = RW1 // DBR

_mesh = plsc.VectorSubcoreMesh(core_axis_name="c", subcore_axis_name="s")
_f32 = jnp.float32
_SC_PARAMS = pltpu.CompilerParams(use_tc_tiling_on_sc=False)


@functools.partial(
    pl.kernel,
    out_type=[jax.ShapeDtypeStruct((NPAD,), _f32),
              jax.ShapeDtypeStruct((NPAD,), _f32)],
    mesh=_mesh,
    compiler_params=_SC_PARAMS,
    scratch_types=[
        pltpu.VMEM((DBR, LANES), jnp.int32),
        pltpu.VMEM((DBR, LANES), jnp.int32),
        pltpu.VMEM((LANES,), _f32),
        pltpu.VMEM((RPT,), _f32),
        pltpu.VMEM_SHARED((NPAD,), _f32),
        pltpu.SemaphoreType.DMA,
        pltpu.SemaphoreType.DMA,
    ],
)
def _sc_deg(dst_hbm, out0_hbm, out1_hbm, blk0, blk1, ones_v, zbuf, hist,
            sem0, sem1):
    c = lax.axis_index("c")
    s = lax.axis_index("s")
    ov = jnp.ones((16,), _f32)
    zv = jnp.zeros((16,), _f32)
    for k in range(LANES // 16):
        ones_v[pl.ds(k * 16, 16)] = ov

    def zrow(i, _):
        zbuf[pl.ds(pl.multiple_of(i * 16, 16), 16)] = zv
        return ()

    lax.fori_loop(0, RPT // 16, zrow, ())
    stripe = pl.multiple_of(s * RPT, 128)
    pltpu.sync_copy(zbuf, hist.at[pl.ds(stripe, RPT)])
    plsc.subcore_barrier()

    base = (c * NS + s) * RW1

    def pair(p, _):
        r0 = base + p * (2 * DBR)
        pltpu.sync_copy(dst_hbm.at[pl.ds(r0, DBR)], blk0)
        d0 = [pltpu.async_copy(ones_v, hist.at[blk0.at[j]], sem0, add=True)
              for j in range(DBR)]
        pltpu.sync_copy(dst_hbm.at[pl.ds(r0 + DBR, DBR)], blk1)
        d1 = [pltpu.async_copy(ones_v, hist.at[blk1.at[j]], sem1, add=True)
              for j in range(DBR)]
        for d in d0:
            d.wait()
        for d in d1:
            d.wait()
        return ()

    lax.fori_loop(0, NDB // 2, pair, ())
    plsc.subcore_barrier()

    @pl.when(c == 0)
    def _():
        pltpu.sync_copy(hist.at[pl.ds(stripe, RPT)],
                        out0_hbm.at[pl.ds(stripe, RPT)])

    @pl.when(c == 1)
    def _():
        pltpu.sync_copy(hist.at[pl.ds(stripe, RPT)],
                        out1_hbm.at[pl.ds(stripe, RPT)])


def _agg_loop(src_hbm, dst_hbm, table, acc, src_blk, dst_blk, rows0, rows1,
              gsem0, gsem1, ssem0, ssem1, base, n_blocks):
    rows = (rows0, rows1)
    gsems = (gsem0, gsem1)
    ssems = (ssem0, ssem1)

    def block(blk, _):
        r0 = base + blk * RPB
        pltpu.sync_copy(src_hbm.at[pl.ds(r0, RPB)], src_blk)
        pltpu.sync_copy(dst_hbm.at[pl.ds(r0, RPB)], dst_blk)

        def fire_gather(i):
            b = i % 2
            return [pltpu.async_copy(table.at[src_blk.at[i * MR + j]],
                                     rows[b].at[pl.ds(j * LANES, LANES)],
                                     gsems[b])
                    for j in range(MR)]

        def fire_scatter(i):
            b = i % 2
            return [pltpu.async_copy(rows[b].at[pl.ds(j * LANES, LANES)],
                                     acc.at[dst_blk.at[i * MR + j]],
                                     ssems[b], add=True)
                    for j in range(MR)]

        gd = {0: fire_gather(0), 1: fire_gather(1)}
        sd = {}
        for d in gd[0]:
            d.wait()
        sd[0] = fire_scatter(0)
        for i in range(2, NMEG):
            for d in sd[i - 2]:
                d.wait()
            gd[i] = fire_gather(i)
            for d in gd[i - 1]:
                d.wait()
            sd[i - 1] = fire_scatter(i - 1)
        for d in gd[NMEG - 1]:
            d.wait()
        sd[NMEG - 1] = fire_scatter(NMEG - 1)
        for d in sd[NMEG - 2]:
            d.wait()
        for d in sd[NMEG - 1]:
            d.wait()
        return ()

    lax.fori_loop(0, n_blocks, block, ())


_AGG_SCRATCH = [
    pltpu.VMEM((RPB, LANES), jnp.int32),
    pltpu.VMEM((RPB, LANES), jnp.int32),
    pltpu.VMEM((MR * LANES, 16), _f32),
    pltpu.VMEM((MR * LANES, 16), _f32),
    pltpu.VMEM_SHARED((NPAD, 16), _f32),
    pltpu.SemaphoreType.DMA,
    pltpu.SemaphoreType.DMA,
    pltpu.SemaphoreType.DMA,
    pltpu.SemaphoreType.DMA,
]


def _acc_init(acc, rows0, rows1, s):
    zv = jnp.zeros((16,), _f32)

    def zrow(i, _):
        rows0[i] = zv
        rows1[i] = zv
        return ()

    lax.fori_loop(0, MR * LANES, zrow, ())
    nfull = RPT // (MR * LANES)
    for k in range(nfull):
        off = pl.multiple_of(s * RPT + k * MR * LANES, 8)
        pltpu.sync_copy(rows0, acc.at[pl.ds(off, MR * LANES)])
    tail = RPT - nfull * MR * LANES
    if tail:
        off = pl.multiple_of(s * RPT + nfull * MR * LANES, 8)
        pltpu.sync_copy(rows0.at[pl.ds(0, tail)], acc.at[pl.ds(off, tail)])
    plsc.subcore_barrier()


def _acc_dump(acc, out0, out1, c, s):
    plsc.subcore_barrier()
    stripe = pl.multiple_of(s * RPT, 128)

    @pl.when(c == 0)
    def _():
        pltpu.sync_copy(acc.at[pl.ds(stripe, RPT)],
                        out0.at[pl.ds(stripe, RPT)])

    @pl.when(c == 1)
    def _():
        pltpu.sync_copy(acc.at[pl.ds(stripe, RPT)],
                        out1.at[pl.ds(stripe, RPT)])


_AGG_OUT = [jax.ShapeDtypeStruct((NPAD, 16), _f32),
            jax.ShapeDtypeStruct((NPAD, 16), _f32)]


@functools.partial(
    pl.kernel,
    out_type=_AGG_OUT,
    mesh=_mesh,
    compiler_params=_SC_PARAMS,
    scratch_types=_AGG_SCRATCH,
)
def _sc_agg1(src_hbm, dst_hbm, g_hbm, out0_hbm, out1_hbm,
             src_blk, dst_blk, rows0, rows1, acc, gsem0, gsem1, ssem0, ssem1):
    c = lax.axis_index("c")
    s = lax.axis_index("s")
    _acc_init(acc, rows0, rows1, s)
    base = (c * NS + s) * RW1
    _agg_loop(src_hbm, dst_hbm, g_hbm, acc, src_blk, dst_blk, rows0, rows1,
              gsem0, gsem1, ssem0, ssem1, base, NB1)
    _acc_dump(acc, out0_hbm, out1_hbm, c, s)


@functools.partial(
    pl.kernel,
    out_type=_AGG_OUT,
    mesh=_mesh,
    compiler_params=_SC_PARAMS,
    scratch_types=_AGG_SCRATCH,
)
def _sc_agg2(src_hbm, dst_hbm, ga_hbm, gb_hbm, out0_hbm, out1_hbm,
             src_blk, dst_blk, rows0, rows1, acc, gsem0, gsem1, ssem0, ssem1):
    c = lax.axis_index("c")
    s = lax.axis_index("s")
    _acc_init(acc, rows0, rows1, s)
    base = s * RW2

    @pl.when(c == 0)
    def _():
        _agg_loop(src_hbm, dst_hbm, ga_hbm, acc, src_blk, dst_blk,
                  rows0, rows1, gsem0, gsem1, ssem0, ssem1, base, NB2)

    @pl.when(c == 1)
    def _():
        _agg_loop(src_hbm, dst_hbm, gb_hbm, acc, src_blk, dst_blk,
                  rows0, rows1, gsem0, gsem1, ssem0, ssem1, base, NB2)

    _acc_dump(acc, out0_hbm, out1_hbm, c, s)


BR = 3136
_GRID = NPAD // BR


def _row_spec(width):
    return pl.BlockSpec((BR, width), lambda i: (i, 0))


def _whole_spec(shape):
    return pl.BlockSpec(shape, lambda i: (0,) * len(shape))


def _tca_body(x_ref, h0_ref, h1_ref, w_ref, g_ref, dinv_ref):
    deg = h0_ref[...] + h1_ref[...] + 1.0
    dinv = lax.rsqrt(deg)
    h = lax.dot_general(x_ref[...], w_ref[...], (((1,), (0,)), ((), ())),
                        preferred_element_type=_f32)
    g_ref[...] = h * dinv
    dinv_ref[...] = dinv


_tc_a = pl.pallas_call(
    _tca_body,
    grid=(_GRID,),
    in_specs=[_row_spec(3), _row_spec(1), _row_spec(1), _whole_spec((3, 16))],
    out_specs=[_row_spec(16), _row_spec(1)],
    out_shape=[jax.ShapeDtypeStruct((NPAD, 16), _f32),
               jax.ShapeDtypeStruct((NPAD, 1), _f32)],
)


def _tcb_body(p0_ref, p1_ref, g1_ref, dinv_ref, b1_ref, w_ref,
              ga_ref, gb_ref):
    dinv = dinv_ref[...]
    h = (p0_ref[...] + p1_ref[...] + g1_ref[...]) * dinv + b1_ref[...]
    h = jnp.maximum(h, 0.0)
    g2 = lax.dot_general(h, w_ref[...], (((1,), (0,)), ((), ())),
                         preferred_element_type=_f32) * dinv
    ga_ref[...] = g2[:, :16]
    gb_ref[...] = g2[:, 16:]


_tc_b = pl.pallas_call(
    _tcb_body,
    grid=(_GRID,),
    in_specs=[_row_spec(16), _row_spec(16), _row_spec(16), _row_spec(1),
              _whole_spec((1, 16)), _whole_spec((16, 32))],
    out_specs=[_row_spec(16), _row_spec(16)],
    out_shape=[jax.ShapeDtypeStruct((NPAD, 16), _f32),
               jax.ShapeDtypeStruct((NPAD, 16), _f32)],
)


def _tcc_body(qa_ref, qb_ref, ga_ref, gb_ref, dinv_ref, b2_ref, out_ref):
    dinv = dinv_ref[...]
    z = jnp.concatenate([qa_ref[...] + ga_ref[...],
                         qb_ref[...] + gb_ref[...]], axis=1)
    z = z * dinv + b2_ref[...]
    z = z - jnp.max(z, axis=1, keepdims=True)
    out_ref[...] = z - jnp.log(jnp.sum(jnp.exp(z), axis=1, keepdims=True))


_tc_c = pl.pallas_call(
    _tcc_body,
    grid=(_GRID,),
    in_specs=[_row_spec(16), _row_spec(16), _row_spec(16), _row_spec(16),
              _row_spec(1), _whole_spec((1, 32))],
    out_specs=_row_spec(32),
    out_shape=jax.ShapeDtypeStruct((NPAD, 32), _f32),
)


def kernel(x, edge_index, W1, b1, W2, b2):
    x = x.astype(_f32)
    src = edge_index[0]
    dst = edge_index[1]
    pad = jnp.full((EPAD - E,), N, jnp.int32)
    srcp = jnp.concatenate([src, pad]).reshape(ROWS_E, LANES)
    dstp = jnp.concatenate([dst, pad]).reshape(ROWS_E, LANES)
    xp = jnp.pad(x, ((0, NPAD - N), (0, 0)))

    h0, h1 = _sc_deg(dstp)
    g1, dinv = _tc_a(xp, h0[:, None], h1[:, None], W1)
    p0, p1 = _sc_agg1(srcp, dstp, g1)
    ga, gb = _tc_b(p0, p1, g1, dinv, b1[None, :], W2)
    qa, qb = _sc_agg2(srcp, dstp, ga, gb)
    out = _tc_c(qa, qb, ga, gb, dinv, b2[None, :])
    return out[:N]


# speedup vs baseline: 1.5915x; 1.5915x over previous
import functools

import jax
import jax.numpy as jnp
from jax import lax
from jax.experimental import pallas as pl
from jax.experimental.pallas import tpu as pltpu
from jax.experimental.pallas import tpu_sc as plsc

N = 100000
E = 3200000

NC = 2
NS = 16
LANES = 128
MR = 4
NMEG = 8
RPB = NMEG * MR

NPAD = 106496
RPT = NPAD // NS
EPAD = 3276800
ROWS_E = EPAD // LANES
RW1 = ROWS_E // (NC * NS)
RW2 = ROWS_E // NS
NB1 = RW1 // RPB
NB2 = RW2 // RPB
DBR = 40
NDB = RW1 // DBR

_mesh = plsc.VectorSubcoreMesh(core_axis_name="c", subcore_axis_name="s")
_f32 = jnp.float32
_SC_PARAMS = pltpu.CompilerParams(use_tc_tiling_on_sc=False)


@functools.partial(
    pl.kernel,
    out_type=[jax.ShapeDtypeStruct((NPAD,), _f32),
              jax.ShapeDtypeStruct((NPAD,), _f32)],
    mesh=_mesh,
    compiler_params=_SC_PARAMS,
    scratch_types=[
        pltpu.VMEM((DBR, LANES), jnp.int32),
        pltpu.VMEM((DBR, LANES), jnp.int32),
        pltpu.VMEM((LANES,), _f32),
        pltpu.VMEM((RPT,), _f32),
        pltpu.VMEM_SHARED((NPAD,), _f32),
        pltpu.SemaphoreType.DMA,
        pltpu.SemaphoreType.DMA,
    ],
)
def _sc_deg(dst_hbm, out0_hbm, out1_hbm, blk0, blk1, ones_v, zbuf, hist,
            sem0, sem1):
    c = lax.axis_index("c")
    s = lax.axis_index("s")
    ov = jnp.ones((16,), _f32)
    zv = jnp.zeros((16,), _f32)
    for k in range(LANES // 16):
        ones_v[pl.ds(k * 16, 16)] = ov

    def zrow(i, _):
        zbuf[pl.ds(pl.multiple_of(i * 16, 16), 16)] = zv
        return ()

    lax.fori_loop(0, RPT // 16, zrow, ())
    stripe = pl.multiple_of(s * RPT, 128)
    pltpu.sync_copy(zbuf, hist.at[pl.ds(stripe, RPT)])
    plsc.subcore_barrier()

    base = (c * NS + s) * RW1

    def pair(p, _):
        r0 = base + p * (2 * DBR)
        pltpu.sync_copy(dst_hbm.at[pl.ds(r0, DBR)], blk0)
        d0 = [pltpu.async_copy(ones_v, hist.at[blk0.at[j]], sem0, add=True)
              for j in range(DBR)]
        pltpu.sync_copy(dst_hbm.at[pl.ds(r0 + DBR, DBR)], blk1)
        d1 = [pltpu.async_copy(ones_v, hist.at[blk1.at[j]], sem1, add=True)
              for j in range(DBR)]
        for d in d0:
            d.wait()
        for d in d1:
            d.wait()
        return ()

    lax.fori_loop(0, NDB // 2, pair, ())
    plsc.subcore_barrier()

    @pl.when(c == 0)
    def _():
        pltpu.sync_copy(hist.at[pl.ds(stripe, RPT)],
                        out0_hbm.at[pl.ds(stripe, RPT)])

    @pl.when(c == 1)
    def _():
        pltpu.sync_copy(hist.at[pl.ds(stripe, RPT)],
                        out1_hbm.at[pl.ds(stripe, RPT)])


def _agg_loop(src_hbm, dst_hbm, table, acc, src_blk, dst_blk, rows0, rows1,
              gsem0, gsem1, ssem0, ssem1, base, n_blocks):
    rows = (rows0, rows1)
    gsems = (gsem0, gsem1)
    ssems = (ssem0, ssem1)

    def block(blk, _):
        r0 = base + blk * RPB
        pltpu.sync_copy(src_hbm.at[pl.ds(r0, RPB)], src_blk)
        pltpu.sync_copy(dst_hbm.at[pl.ds(r0, RPB)], dst_blk)

        def fire_gather(i):
            b = i % 2
            return [pltpu.async_copy(table.at[src_blk.at[i * MR + j]],
                                     rows[b].at[pl.ds(j * LANES, LANES)],
                                     gsems[b])
                    for j in range(MR)]

        def fire_scatter(i):
            b = i % 2
            return [pltpu.async_copy(rows[b].at[pl.ds(j * LANES, LANES)],
                                     acc.at[dst_blk.at[i * MR + j]],
                                     ssems[b], add=True)
                    for j in range(MR)]

        def drain(ds):
            for d in ds:
                d.wait()

        gd = {0: fire_gather(0), 1: fire_gather(1)}
        sd = {}
        drain(gd[0])
        sd[0] = fire_scatter(0)
        for i in range(2, NMEG):
            drain(sd[i - 2])
            gd[i] = fire_gather(i)
            drain(gd[i - 1])
            sd[i - 1] = fire_scatter(i - 1)
        drain(gd[NMEG - 1])
        sd[NMEG - 1] = fire_scatter(NMEG - 1)
        drain(sd[NMEG - 2])
        drain(sd[NMEG - 1])
        return ()

    lax.fori_loop(0, n_blocks, block, ())


_AGG_SCRATCH = [
    pltpu.VMEM((RPB, LANES), jnp.int32),
    pltpu.VMEM((RPB, LANES), jnp.int32),
    pltpu.VMEM((MR * LANES, 16), _f32),
    pltpu.VMEM((MR * LANES, 16), _f32),
    pltpu.VMEM_SHARED((NPAD, 16), _f32),
    pltpu.SemaphoreType.DMA,
    pltpu.SemaphoreType.DMA,
    pltpu.SemaphoreType.DMA,
    pltpu.SemaphoreType.DMA,
]


def _acc_init(acc, rows0, rows1, s):
    zv = jnp.zeros((16,), _f32)

    def zrow(i, _):
        rows0[i] = zv
        rows1[i] = zv
        return ()

    lax.fori_loop(0, MR * LANES, zrow, ())
    nfull = RPT // (MR * LANES)
    for k in range(nfull):
        off = pl.multiple_of(s * RPT + k * MR * LANES, 8)
        pltpu.sync_copy(rows0, acc.at[pl.ds(off, MR * LANES)])
    tail = RPT - nfull * MR * LANES
    if tail:
        off = pl.multiple_of(s * RPT + nfull * MR * LANES, 8)
        pltpu.sync_copy(rows0.at[pl.ds(0, tail)], acc.at[pl.ds(off, tail)])
    plsc.subcore_barrier()


def _acc_dump(acc, out0, out1, c, s):
    plsc.subcore_barrier()
    stripe = pl.multiple_of(s * RPT, 128)

    @pl.when(c == 0)
    def _():
        pltpu.sync_copy(acc.at[pl.ds(stripe, RPT)],
                        out0.at[pl.ds(stripe, RPT)])

    @pl.when(c == 1)
    def _():
        pltpu.sync_copy(acc.at[pl.ds(stripe, RPT)],
                        out1.at[pl.ds(stripe, RPT)])


_AGG_OUT = [jax.ShapeDtypeStruct((NPAD, 16), _f32),
            jax.ShapeDtypeStruct((NPAD, 16), _f32)]


@functools.partial(
    pl.kernel,
    out_type=_AGG_OUT,
    mesh=_mesh,
    compiler_params=_SC_PARAMS,
    scratch_types=_AGG_SCRATCH,
)
def _sc_agg1(src_hbm, dst_hbm, g_hbm, out0_hbm, out1_hbm,
             src_blk, dst_blk, rows0, rows1, acc, gsem0, gsem1, ssem0, ssem1):
    c = lax.axis_index("c")
    s = lax.axis_index("s")
    _acc_init(acc, rows0, rows1, s)
    base = (c * NS + s) * RW1
    _agg_loop(src_hbm, dst_hbm, g_hbm, acc, src_blk, dst_blk, rows0, rows1,
              gsem0, gsem1, ssem0, ssem1, base, NB1)
    _acc_dump(acc, out0_hbm, out1_hbm, c, s)


@functools.partial(
    pl.kernel,
    out_type=_AGG_OUT,
    mesh=_mesh,
    compiler_params=_SC_PARAMS,
    scratch_types=_AGG_SCRATCH,
)
def _sc_agg2(src_hbm, dst_hbm, ga_hbm, gb_hbm, out0_hbm, out1_hbm,
             src_blk, dst_blk, rows0, rows1, acc, gsem0, gsem1, ssem0, ssem1):
    c = lax.axis_index("c")
    s = lax.axis_index("s")
    _acc_init(acc, rows0, rows1, s)
    base = s * RW2

    @pl.when(c == 0)
    def _():
        _agg_loop(src_hbm, dst_hbm, ga_hbm, acc, src_blk, dst_blk,
                  rows0, rows1, gsem0, gsem1, ssem0, ssem1, base, NB2)

    @pl.when(c == 1)
    def _():
        _agg_loop(src_hbm, dst_hbm, gb_hbm, acc, src_blk, dst_blk,
                  rows0, rows1, gsem0, gsem1, ssem0, ssem1, base, NB2)

    _acc_dump(acc, out0_hbm, out1_hbm, c, s)


BR = 3328
_GRID = NPAD // BR


def _row_spec(width):
    return pl.BlockSpec((BR, width), lambda i: (i, 0))


def _whole_spec(shape):
    return pl.BlockSpec(shape, lambda i: (0,) * len(shape))


def _tca_body(x_ref, h0_ref, h1_ref, w_ref, g_ref, dinv_ref):
    deg = h0_ref[...] + h1_ref[...] + 1.0
    dinv = lax.rsqrt(deg)
    h = lax.dot_general(x_ref[...], w_ref[...], (((1,), (0,)), ((), ())),
                        preferred_element_type=_f32)
    g_ref[...] = h * dinv
    dinv_ref[...] = dinv


_tc_a = pl.pallas_call(
    _tca_body,
    grid=(_GRID,),
    in_specs=[_row_spec(3), _row_spec(1), _row_spec(1), _whole_spec((3, 16))],
    out_specs=[_row_spec(16), _row_spec(1)],
    out_shape=[jax.ShapeDtypeStruct((NPAD, 16), _f32),
               jax.ShapeDtypeStruct((NPAD, 1), _f32)],
)


def _tcb_body(p0_ref, p1_ref, g1_ref, dinv_ref, b1_ref, w_ref,
              ga_ref, gb_ref):
    dinv = dinv_ref[...]
    h = (p0_ref[...] + p1_ref[...] + g1_ref[...]) * dinv + b1_ref[...]
    h = jnp.maximum(h, 0.0)
    g2 = lax.dot_general(h, w_ref[...], (((1,), (0,)), ((), ())),
                         preferred_element_type=_f32) * dinv
    ga_ref[...] = g2[:, :16]
    gb_ref[...] = g2[:, 16:]


_tc_b = pl.pallas_call(
    _tcb_body,
    grid=(_GRID,),
    in_specs=[_row_spec(16), _row_spec(16), _row_spec(16), _row_spec(1),
              _whole_spec((1, 16)), _whole_spec((16, 32))],
    out_specs=[_row_spec(16), _row_spec(16)],
    out_shape=[jax.ShapeDtypeStruct((NPAD, 16), _f32),
               jax.ShapeDtypeStruct((NPAD, 16), _f32)],
)


def _tcc_body(qa_ref, qb_ref, ga_ref, gb_ref, dinv_ref, b2_ref, out_ref):
    dinv = dinv_ref[...]
    z = jnp.concatenate([qa_ref[...] + ga_ref[...],
                         qb_ref[...] + gb_ref[...]], axis=1)
    z = z * dinv + b2_ref[...]
    z = z - jnp.max(z, axis=1, keepdims=True)
    out_ref[...] = z - jnp.log(jnp.sum(jnp.exp(z), axis=1, keepdims=True))


_tc_c = pl.pallas_call(
    _tcc_body,
    grid=(_GRID,),
    in_specs=[_row_spec(16), _row_spec(16), _row_spec(16), _row_spec(16),
              _row_spec(1), _whole_spec((1, 32))],
    out_specs=_row_spec(32),
    out_shape=jax.ShapeDtypeStruct((NPAD, 32), _f32),
)


def kernel(x, edge_index, W1, b1, W2, b2):
    x = x.astype(_f32)
    src = edge_index[0]
    dst = edge_index[1]
    pad = N + jnp.arange(EPAD - E, dtype=jnp.int32) % (NPAD - N)
    srcp = jnp.concatenate([src, pad]).reshape(ROWS_E, LANES)
    dstp = jnp.concatenate([dst, pad]).reshape(ROWS_E, LANES)
    xp = jnp.pad(x, ((0, NPAD - N), (0, 0)))

    h0, h1 = _sc_deg(dstp)
    g1, dinv = _tc_a(xp, h0[:, None], h1[:, None], W1)
    p0, p1 = _sc_agg1(srcp, dstp, g1)
    ga, gb = _tc_b(p0, p1, g1, dinv, b1[None, :], W2)
    qa, qb = _sc_agg2(srcp, dstp, ga, gb)
    out = _tc_c(qa, qb, ga, gb, dinv, b2[None, :])
    return out[:N]
